# baseline (speedup 1.0000x reference)
import functools

import jax
import jax.numpy as jnp
from jax import lax
from jax.experimental import pallas as pl
from jax.experimental.pallas import tpu as pltpu
from jax.experimental.pallas import tpu_sc as plsc

N = 10000
E = 320000
D = 128
H = 128
CW = 16
NC = 2
NS = 16
NW = NC * NS
EPW = E // NW
K = 80
NB = EPW // K
NA = 10240
RPT = NA // NS
CB = 25
NCH = NB // CB


def _sc_segment_sum(x, edge_index, z128, z16, ones16):
    mesh = plsc.VectorSubcoreMesh(core_axis_name="c", subcore_axis_name="s")

    @functools.partial(
        pl.kernel,
        out_type=(jax.ShapeDtypeStruct((NC, NA, D), jnp.float32),
                  jax.ShapeDtypeStruct((NC, NA, CW), jnp.float32)),
        mesh=mesh,
        scratch_types=[
            pltpu.VMEM_SHARED((NA, D), jnp.float32),
            pltpu.VMEM_SHARED((NA, CW), jnp.float32),
            pltpu.VMEM((CB * K,), jnp.int32),
            pltpu.VMEM((CB * K,), jnp.int32),
            pltpu.VMEM((CB * K,), jnp.int32),
            pltpu.VMEM((CB * K,), jnp.int32),
            pltpu.VMEM((K, D), jnp.float32),
            pltpu.VMEM((K, D), jnp.float32),
            pltpu.VMEM((K, CW), jnp.float32),
            pltpu.VMEM((K, CW), jnp.float32),
            pltpu.SemaphoreType.DMA,
            pltpu.SemaphoreType.DMA,
            pltpu.SemaphoreType.DMA,
            pltpu.SemaphoreType.DMA,
            pltpu.SemaphoreType.DMA,
        ],
        compiler_params=pltpu.CompilerParams(use_tc_tiling_on_sc=False),
    )
    def k(x_hbm, ei_hbm, z128_hbm, z16_hbm, ones_hbm, outd_hbm, outc_hbm,
          accd, accc, src_a, dst_a, src_b, dst_b, rows0, rows1, zc_v, ones_v,
          gsem0, gsem1, isem, wsem, osem):
        cid = lax.axis_index("c")
        sid = lax.axis_index("s")
        wid = sid * NC + cid
        srcs = (src_a, src_b)
        dsts = (dst_a, dst_b)
        ebase = wid * EPW

        pltpu.sync_copy(z128_hbm, rows0)
        pltpu.sync_copy(z16_hbm, zc_v)
        pltpu.sync_copy(ones_hbm, ones_v)
        row0 = sid * RPT
        zh = []
        for t in range(RPT // K):
            zh.append(pltpu.async_copy(
                rows0, accd.at[pl.ds(row0 + t * K, K)], wsem))
            zh.append(pltpu.async_copy(
                zc_v, accc.at[pl.ds(row0 + t * K, K)], wsem))
        ih = [pltpu.async_copy(ei_hbm.at[0, pl.ds(ebase, CB * K)], src_a,
                               isem),
              pltpu.async_copy(ei_hbm.at[1, pl.ds(ebase, CB * K)], dst_a,
                               isem)]
        for h in zh:
            h.wait()
        for h in ih:
            h.wait()
        pltpu.async_copy(x_hbm.at[src_a.at[pl.ds(0, K)]], rows0, gsem0)

        plsc.subcore_barrier()

        for c in range(NCH):
            src_v = srcs[c % 2]
            dst_v = dsts[c % 2]
            if c % 2 == 0:
                re, ge = rows0, gsem0
                ro, go = rows1, gsem1
            else:
                re, ge = rows1, gsem1
                ro, go = rows0, gsem0
            if c + 1 < NCH:
                nsrc = srcs[(c + 1) % 2]
                ndst = dsts[(c + 1) % 2]
                off = ebase + (c + 1) * CB * K
                ih = [pltpu.async_copy(ei_hbm.at[0, pl.ds(off, CB * K)],
                                       nsrc, isem),
                      pltpu.async_copy(ei_hbm.at[1, pl.ds(off, CB * K)],
                                       ndst, isem)]

            def cnt_wait():
                pltpu.make_async_copy(
                    ones_v, accc.at[dst_v.at[pl.ds(0, K)]], osem).wait()

            def pair(j, carry):
                i0 = 2 * j

                @pl.when(j > 0)
                def _():
                    cnt_wait()
                    cnt_wait()

                pltpu.async_copy(
                    x_hbm.at[src_v.at[pl.ds((i0 + 1) * K, K)]], ro, go)
                pltpu.make_async_copy(
                    x_hbm.at[src_v.at[pl.ds(0, K)]], re, ge).wait()
                pltpu.sync_copy(re, accd.at[dst_v.at[pl.ds(i0 * K, K)]],
                                add=True)
                pltpu.async_copy(ones_v,
                                 accc.at[dst_v.at[pl.ds(i0 * K, K)]],
                                 osem, add=True)
                pltpu.async_copy(
                    x_hbm.at[src_v.at[pl.ds((i0 + 2) * K, K)]], re, ge)
                pltpu.make_async_copy(
                    x_hbm.at[src_v.at[pl.ds(0, K)]], ro, go).wait()
                pltpu.sync_copy(ro,
                                accd.at[dst_v.at[pl.ds((i0 + 1) * K, K)]],
                                add=True)
                pltpu.async_copy(ones_v,
                                 accc.at[dst_v.at[pl.ds((i0 + 1) * K, K)]],
                                 osem, add=True)
                return carry

            lax.fori_loop(0, CB // 2, pair, 0)
            cnt_wait()
            cnt_wait()

            if c + 1 < NCH:
                for h in ih:
                    h.wait()
            pltpu.make_async_copy(
                x_hbm.at[src_v.at[pl.ds(0, K)]], re, ge).wait()
            if c + 1 < NCH:
                pltpu.async_copy(x_hbm.at[nsrc.at[pl.ds(0, K)]], ro, go)
            pltpu.sync_copy(re, accd.at[dst_v.at[pl.ds((CB - 1) * K, K)]],
                            add=True)
            pltpu.async_copy(ones_v,
                             accc.at[dst_v.at[pl.ds((CB - 1) * K, K)]],
                             osem, add=True)
            cnt_wait()

        plsc.subcore_barrier()

        wh = [None, None, None, None]
        dbufs = (rows0, rows1)
        cbufs = (zc_v, ones_v)
        for t in range(RPT // K):
            b = t % 2
            r = row0 + t * K
            if wh[b] is not None:
                wh[b].wait()
            pltpu.sync_copy(accd.at[pl.ds(r, K)], dbufs[b])
            wh[b] = pltpu.async_copy(dbufs[b], outd_hbm.at[cid, pl.ds(r, K)],
                                     wsem)
            if wh[2 + b] is not None:
                wh[2 + b].wait()
            pltpu.sync_copy(accc.at[pl.ds(r, K)], cbufs[b])
            wh[2 + b] = pltpu.async_copy(cbufs[b],
                                         outc_hbm.at[cid, pl.ds(r, K)], wsem)
        for h in wh:
            h.wait()

    return k(x, edge_index, z128, z16, ones16)


def _tc_dense(accd, accc, x, W_l, b_l, W_r):
    BN = 5000
    grid = (N // BN,)

    def body(ad_ref, ac_ref, x_ref, wl_ref, b_ref, wr_ref, o_ref):
        s = ad_ref[0] + ad_ref[1]
        c = jnp.sum(ac_ref[0] + ac_ref[1], axis=1, keepdims=True)
        mean = s / jnp.maximum(c, 1.0)
        out = lax.dot_general(mean, wl_ref[...], (((1,), (1,)), ((), ())),
                              preferred_element_type=jnp.float32)
        out = out + lax.dot_general(x_ref[...], wr_ref[...],
                                    (((1,), (1,)), ((), ())),
                                    preferred_element_type=jnp.float32)
        o_ref[...] = jnp.maximum(out + b_ref[...], 0.0)

    return pl.pallas_call(
        body,
        grid=grid,
        in_specs=[
            pl.BlockSpec((NC, BN, D), lambda i: (0, i, 0)),
            pl.BlockSpec((NC, BN, CW), lambda i: (0, i, 0)),
            pl.BlockSpec((BN, D), lambda i: (i, 0)),
            pl.BlockSpec((H, D), lambda i: (0, 0)),
            pl.BlockSpec((1, H), lambda i: (0, 0)),
            pl.BlockSpec((H, D), lambda i: (0, 0)),
        ],
        out_specs=pl.BlockSpec((BN, H), lambda i: (i, 0)),
        out_shape=jax.ShapeDtypeStruct((N, H), jnp.float32),
    )(accd, accc, x, W_l, b_l, W_r)


def kernel(x, edge_weight, W_l, b_l, W_r, edge_index):
    del edge_weight
    z128 = jnp.zeros((K, D), jnp.float32)
    z16 = jnp.zeros((K, CW), jnp.float32)
    ones16 = jnp.zeros((K, CW), jnp.float32).at[:, 0].set(1.0)
    accd, accc = _sc_segment_sum(x, edge_index, z128, z16, ones16)
    return _tc_dense(accd, accc, x, W_l, b_l.reshape(1, H), W_r)


# speedup vs baseline: 1.0015x; 1.0015x over previous
import functools

import jax
import jax.numpy as jnp
from jax import lax
from jax.experimental import pallas as pl
from jax.experimental.pallas import tpu as pltpu
from jax.experimental.pallas import tpu_sc as plsc

N = 10000
E = 320000
D = 128
H = 128
CW = 16
NC = 2
NS = 16
NW = NC * NS
EPW = E // NW
K = 80
NB = EPW // K
NA = 10240
RPT = NA // NS
CB = 25
NCH = NB // CB


def _sc_segment_sum(x, edge_index, z128, z16, ones16):
    mesh = plsc.VectorSubcoreMesh(core_axis_name="c", subcore_axis_name="s")

    @functools.partial(
        pl.kernel,
        out_type=(jax.ShapeDtypeStruct((NC, NA, D), jnp.float32),
                  jax.ShapeDtypeStruct((NC, NA, CW), jnp.float32)),
        mesh=mesh,
        scratch_types=[
            pltpu.VMEM_SHARED((NA, D), jnp.float32),
            pltpu.VMEM_SHARED((NA, CW), jnp.float32),
            pltpu.VMEM((CB * K,), jnp.int32),
            pltpu.VMEM((CB * K,), jnp.int32),
            pltpu.VMEM((CB * K,), jnp.int32),
            pltpu.VMEM((CB * K,), jnp.int32),
            pltpu.VMEM((K, D), jnp.float32),
            pltpu.VMEM((K, D), jnp.float32),
            pltpu.VMEM((K, CW), jnp.float32),
            pltpu.VMEM((K, CW), jnp.float32),
            pltpu.SemaphoreType.DMA,
            pltpu.SemaphoreType.DMA,
            pltpu.SemaphoreType.DMA,
            pltpu.SemaphoreType.DMA,
            pltpu.SemaphoreType.DMA,
        ],
        compiler_params=pltpu.CompilerParams(use_tc_tiling_on_sc=False,
                                             needs_layout_passes=True),
    )
    def k(x_hbm, ei_hbm, z128_hbm, z16_hbm, ones_hbm, outd_hbm, outc_hbm,
          accd, accc, src_a, dst_a, src_b, dst_b, rows0, rows1, zc_v, ones_v,
          gsem0, gsem1, isem, wsem, osem):
        cid = lax.axis_index("c")
        sid = lax.axis_index("s")
        wid = sid * NC + cid
        srcs = (src_a, src_b)
        dsts = (dst_a, dst_b)
        ebase = wid * EPW

        pltpu.sync_copy(z128_hbm, rows0)
        pltpu.sync_copy(z16_hbm, zc_v)
        pltpu.sync_copy(ones_hbm, ones_v)
        row0 = sid * RPT
        zh = []
        for t in range(RPT // K):
            zh.append(pltpu.async_copy(
                rows0, accd.at[pl.ds(row0 + t * K, K)], wsem))
            zh.append(pltpu.async_copy(
                zc_v, accc.at[pl.ds(row0 + t * K, K)], wsem))
        ih = [pltpu.async_copy(ei_hbm.at[0, pl.ds(ebase, CB * K)], src_a,
                               isem),
              pltpu.async_copy(ei_hbm.at[1, pl.ds(ebase, CB * K)], dst_a,
                               isem)]
        for h in zh:
            h.wait()
        for h in ih:
            h.wait()
        pltpu.async_copy(x_hbm.at[src_a.at[pl.ds(0, K)]], rows0, gsem0)

        plsc.subcore_barrier()

        for c in range(NCH):
            src_v = srcs[c % 2]
            dst_v = dsts[c % 2]
            if c % 2 == 0:
                re, ge = rows0, gsem0
                ro, go = rows1, gsem1
            else:
                re, ge = rows1, gsem1
                ro, go = rows0, gsem0
            if c + 1 < NCH:
                nsrc = srcs[(c + 1) % 2]
                ndst = dsts[(c + 1) % 2]
                off = ebase + (c + 1) * CB * K
                ih = [pltpu.async_copy(ei_hbm.at[0, pl.ds(off, CB * K)],
                                       nsrc, isem),
                      pltpu.async_copy(ei_hbm.at[1, pl.ds(off, CB * K)],
                                       ndst, isem)]

            def cnt_wait():
                pltpu.make_async_copy(
                    ones_v, accc.at[dst_v.at[pl.ds(0, K)]], osem).wait()

            def pair(j, carry):
                i0 = 2 * j

                @pl.when(j > 0)
                def _():
                    cnt_wait()
                    cnt_wait()

                pltpu.async_copy(
                    x_hbm.at[src_v.at[pl.ds((i0 + 1) * K, K)]], ro, go)
                pltpu.make_async_copy(
                    x_hbm.at[src_v.at[pl.ds(0, K)]], re, ge).wait()
                pltpu.sync_copy(re, accd.at[dst_v.at[pl.ds(i0 * K, K)]],
                                add=True)
                pltpu.async_copy(ones_v,
                                 accc.at[dst_v.at[pl.ds(i0 * K, K)]],
                                 osem, add=True)
                pltpu.async_copy(
                    x_hbm.at[src_v.at[pl.ds((i0 + 2) * K, K)]], re, ge)
                pltpu.make_async_copy(
                    x_hbm.at[src_v.at[pl.ds(0, K)]], ro, go).wait()
                pltpu.sync_copy(ro,
                                accd.at[dst_v.at[pl.ds((i0 + 1) * K, K)]],
                                add=True)
                pltpu.async_copy(ones_v,
                                 accc.at[dst_v.at[pl.ds((i0 + 1) * K, K)]],
                                 osem, add=True)
                return carry

            lax.fori_loop(0, CB // 2, pair, 0)
            cnt_wait()
            cnt_wait()

            if c + 1 < NCH:
                for h in ih:
                    h.wait()
            pltpu.make_async_copy(
                x_hbm.at[src_v.at[pl.ds(0, K)]], re, ge).wait()
            if c + 1 < NCH:
                pltpu.async_copy(x_hbm.at[nsrc.at[pl.ds(0, K)]], ro, go)
            pltpu.sync_copy(re, accd.at[dst_v.at[pl.ds((CB - 1) * K, K)]],
                            add=True)
            pltpu.async_copy(ones_v,
                             accc.at[dst_v.at[pl.ds((CB - 1) * K, K)]],
                             osem, add=True)
            cnt_wait()

        plsc.subcore_barrier()

        wh = [None, None, None, None]
        dbufs = (rows0, rows1)
        cbufs = (zc_v, ones_v)
        for t in range(RPT // K):
            b = t % 2
            r = row0 + t * K
            if wh[b] is not None:
                wh[b].wait()
            pltpu.sync_copy(accd.at[pl.ds(r, K)], dbufs[b])
            wh[b] = pltpu.async_copy(dbufs[b], outd_hbm.at[cid, pl.ds(r, K)],
                                     wsem)
            if wh[2 + b] is not None:
                wh[2 + b].wait()
            pltpu.sync_copy(accc.at[pl.ds(r, K)], cbufs[b])
            wh[2 + b] = pltpu.async_copy(cbufs[b],
                                         outc_hbm.at[cid, pl.ds(r, K)], wsem)
        for h in wh:
            h.wait()

    return k(x, edge_index, z128, z16, ones16)


def _tc_dense(accd, accc, x, W_l, b_l, W_r):
    BN = 2000
    grid = (N // BN,)

    def body(ad_ref, ac_ref, x_ref, wl_ref, b_ref, wr_ref, o_ref):
        s = ad_ref[0] + ad_ref[1]
        c = jnp.sum(ac_ref[0] + ac_ref[1], axis=1, keepdims=True)
        mean = s / jnp.maximum(c, 1.0)
        out = lax.dot_general(mean, wl_ref[...], (((1,), (1,)), ((), ())),
                              preferred_element_type=jnp.float32)
        out = out + lax.dot_general(x_ref[...], wr_ref[...],
                                    (((1,), (1,)), ((), ())),
                                    preferred_element_type=jnp.float32)
        o_ref[...] = jnp.maximum(out + b_ref[...], 0.0)

    return pl.pallas_call(
        body,
        grid=grid,
        in_specs=[
            pl.BlockSpec((NC, BN, D), lambda i: (0, i, 0)),
            pl.BlockSpec((NC, BN, CW), lambda i: (0, i, 0)),
            pl.BlockSpec((BN, D), lambda i: (i, 0)),
            pl.BlockSpec((H, D), lambda i: (0, 0)),
            pl.BlockSpec((1, H), lambda i: (0, 0)),
            pl.BlockSpec((H, D), lambda i: (0, 0)),
        ],
        out_specs=pl.BlockSpec((BN, H), lambda i: (i, 0)),
        out_shape=jax.ShapeDtypeStruct((N, H), jnp.float32),
    )(accd, accc, x, W_l, b_l, W_r)


def kernel(x, edge_weight, W_l, b_l, W_r, edge_index):
    del edge_weight
    z128 = jnp.zeros((K, D), jnp.float32)
    z16 = jnp.zeros((K, CW), jnp.float32)
    ones16 = jnp.zeros((K, CW), jnp.float32).at[:, 0].set(1.0)
    accd, accc = _sc_segment_sum(x, edge_index, z128, z16, ones16)
    return _tc_dense(accd, accc, x, W_l, b_l.reshape(1, H), W_r)


# speedup vs baseline: 1.0266x; 1.0250x over previous
import functools

import jax
import jax.numpy as jnp
from jax import lax
from jax.experimental import pallas as pl
from jax.experimental.pallas import tpu as pltpu
from jax.experimental.pallas import tpu_sc as plsc

N = 10000
E = 320000
D = 128
H = 128
CW = 16
NC = 2
NS = 16
NW = NC * NS
EPW = E // NW
K = 80
NB = EPW // K
NA = 10240
RPT = NA // NS
CB = 25
NCH = NB // CB


def _sc_segment_sum(x, edge_index, z128, z16, ones16):
    mesh = plsc.VectorSubcoreMesh(core_axis_name="c", subcore_axis_name="s")

    @functools.partial(
        pl.kernel,
        out_type=(jax.ShapeDtypeStruct((NC, NA, D), jnp.float32),
                  jax.ShapeDtypeStruct((NC, NA, CW), jnp.float32)),
        mesh=mesh,
        scratch_types=[
            pltpu.VMEM_SHARED((NA, D), jnp.float32),
            pltpu.VMEM_SHARED((NA, CW), jnp.float32),
            pltpu.VMEM((CB * K,), jnp.int32),
            pltpu.VMEM((CB * K,), jnp.int32),
            pltpu.VMEM((CB * K,), jnp.int32),
            pltpu.VMEM((CB * K,), jnp.int32),
            pltpu.VMEM((K, D), jnp.float32),
            pltpu.VMEM((K, D), jnp.float32),
            pltpu.VMEM((K, CW), jnp.float32),
            pltpu.VMEM((K, CW), jnp.float32),
            pltpu.SemaphoreType.DMA,
            pltpu.SemaphoreType.DMA,
            pltpu.SemaphoreType.DMA,
            pltpu.SemaphoreType.DMA,
            pltpu.SemaphoreType.DMA,
        ],
        compiler_params=pltpu.CompilerParams(use_tc_tiling_on_sc=False),
    )
    def k(x_hbm, ei_hbm, z128_hbm, z16_hbm, ones_hbm, outd_hbm, outc_hbm,
          accd, accc, src_a, dst_a, src_b, dst_b, rows0, rows1, zc_v, ones_v,
          gsem0, gsem1, isem, wsem, osem):
        cid = lax.axis_index("c")
        sid = lax.axis_index("s")
        wid = sid * NC + cid
        srcs = (src_a, src_b)
        dsts = (dst_a, dst_b)
        ebase = wid * EPW

        pltpu.sync_copy(z128_hbm, rows0)
        pltpu.sync_copy(z16_hbm, zc_v)
        pltpu.sync_copy(ones_hbm, ones_v)
        row0 = sid * RPT
        zh = []
        for t in range(RPT // K):
            zh.append(pltpu.async_copy(
                rows0, accd.at[pl.ds(row0 + t * K, K)], wsem))
            zh.append(pltpu.async_copy(
                zc_v, accc.at[pl.ds(row0 + t * K, K)], wsem))
        ih = [pltpu.async_copy(ei_hbm.at[0, pl.ds(ebase, CB * K)], src_a,
                               isem),
              pltpu.async_copy(ei_hbm.at[1, pl.ds(ebase, CB * K)], dst_a,
                               isem)]
        for h in zh:
            h.wait()
        for h in ih:
            h.wait()
        pltpu.async_copy(x_hbm.at[src_a.at[pl.ds(0, K)]], rows0, gsem0)

        plsc.subcore_barrier()

        for c in range(NCH):
            src_v = srcs[c % 2]
            dst_v = dsts[c % 2]
            if c % 2 == 0:
                re, ge = rows0, gsem0
                ro, go = rows1, gsem1
            else:
                re, ge = rows1, gsem1
                ro, go = rows0, gsem0
            if c + 1 < NCH:
                nsrc = srcs[(c + 1) % 2]
                ndst = dsts[(c + 1) % 2]
                off = ebase + (c + 1) * CB * K
                ih = [pltpu.async_copy(ei_hbm.at[0, pl.ds(off, CB * K)],
                                       nsrc, isem),
                      pltpu.async_copy(ei_hbm.at[1, pl.ds(off, CB * K)],
                                       ndst, isem)]

            def cnt_wait():
                pltpu.make_async_copy(
                    ones_v, accc.at[dst_v.at[pl.ds(0, K)]], osem).wait()

            def pair(j, carry):
                i0 = 2 * j
                pltpu.async_copy(
                    x_hbm.at[src_v.at[pl.ds((i0 + 1) * K, K)]], ro, go)
                pltpu.make_async_copy(
                    x_hbm.at[src_v.at[pl.ds(0, K)]], re, ge).wait()
                pltpu.sync_copy(re, accd.at[dst_v.at[pl.ds(i0 * K, K)]],
                                add=True)

                @pl.when(j > 0)
                def _():
                    cnt_wait()

                pltpu.async_copy(ones_v,
                                 accc.at[dst_v.at[pl.ds(i0 * K, K)]],
                                 osem, add=True)
                pltpu.async_copy(
                    x_hbm.at[src_v.at[pl.ds((i0 + 2) * K, K)]], re, ge)
                pltpu.make_async_copy(
                    x_hbm.at[src_v.at[pl.ds(0, K)]], ro, go).wait()
                pltpu.sync_copy(ro,
                                accd.at[dst_v.at[pl.ds((i0 + 1) * K, K)]],
                                add=True)

                @pl.when(j > 0)
                def _():
                    cnt_wait()

                pltpu.async_copy(ones_v,
                                 accc.at[dst_v.at[pl.ds((i0 + 1) * K, K)]],
                                 osem, add=True)
                return carry

            lax.fori_loop(0, CB // 2, pair, 0)
            cnt_wait()
            cnt_wait()

            if c + 1 < NCH:
                for h in ih:
                    h.wait()
            pltpu.make_async_copy(
                x_hbm.at[src_v.at[pl.ds(0, K)]], re, ge).wait()
            if c + 1 < NCH:
                pltpu.async_copy(x_hbm.at[nsrc.at[pl.ds(0, K)]], ro, go)
            pltpu.sync_copy(re, accd.at[dst_v.at[pl.ds((CB - 1) * K, K)]],
                            add=True)
            pltpu.async_copy(ones_v,
                             accc.at[dst_v.at[pl.ds((CB - 1) * K, K)]],
                             osem, add=True)
            cnt_wait()

        plsc.subcore_barrier()

        wh = [None, None, None, None]
        dbufs = (rows0, rows1)
        cbufs = (zc_v, ones_v)
        for t in range(RPT // K):
            b = t % 2
            r = row0 + t * K
            if wh[b] is not None:
                wh[b].wait()
            pltpu.sync_copy(accd.at[pl.ds(r, K)], dbufs[b])
            wh[b] = pltpu.async_copy(dbufs[b], outd_hbm.at[cid, pl.ds(r, K)],
                                     wsem)
            if wh[2 + b] is not None:
                wh[2 + b].wait()
            pltpu.sync_copy(accc.at[pl.ds(r, K)], cbufs[b])
            wh[2 + b] = pltpu.async_copy(cbufs[b],
                                         outc_hbm.at[cid, pl.ds(r, K)], wsem)
        for h in wh:
            h.wait()

    return k(x, edge_index, z128, z16, ones16)


def _tc_dense(accd, accc, x, W_l, b_l, W_r):
    BN = 2000
    grid = (N // BN,)

    def body(ad_ref, ac_ref, x_ref, wl_ref, b_ref, wr_ref, o_ref):
        s = ad_ref[0] + ad_ref[1]
        c = jnp.sum(ac_ref[0] + ac_ref[1], axis=1, keepdims=True)
        mean = s / jnp.maximum(c, 1.0)
        out = lax.dot_general(mean, wl_ref[...], (((1,), (1,)), ((), ())),
                              preferred_element_type=jnp.float32)
        out = out + lax.dot_general(x_ref[...], wr_ref[...],
                                    (((1,), (1,)), ((), ())),
                                    preferred_element_type=jnp.float32)
        o_ref[...] = jnp.maximum(out + b_ref[...], 0.0)

    return pl.pallas_call(
        body,
        grid=grid,
        in_specs=[
            pl.BlockSpec((NC, BN, D), lambda i: (0, i, 0)),
            pl.BlockSpec((NC, BN, CW), lambda i: (0, i, 0)),
            pl.BlockSpec((BN, D), lambda i: (i, 0)),
            pl.BlockSpec((H, D), lambda i: (0, 0)),
            pl.BlockSpec((1, H), lambda i: (0, 0)),
            pl.BlockSpec((H, D), lambda i: (0, 0)),
        ],
        out_specs=pl.BlockSpec((BN, H), lambda i: (i, 0)),
        out_shape=jax.ShapeDtypeStruct((N, H), jnp.float32),
    )(accd, accc, x, W_l, b_l, W_r)


def kernel(x, edge_weight, W_l, b_l, W_r, edge_index):
    del edge_weight
    z128 = jnp.zeros((K, D), jnp.float32)
    z16 = jnp.zeros((K, CW), jnp.float32)
    ones16 = jnp.zeros((K, CW), jnp.float32).at[:, 0].set(1.0)
    accd, accc = _sc_segment_sum(x, edge_index, z128, z16, ones16)
    return _tc_dense(accd, accc, x, W_l, b_l.reshape(1, H), W_r)
